# Initial kernel scaffold; baseline (speedup 1.0000x reference)
import jax
import jax.numpy as jnp
from jax.experimental import pallas as pl


def kernel(x, W, b, mem):
    raise NotImplementedError("write your pallas kernel here")


# speedup vs baseline: 23.7700x; 23.7700x over previous
import functools
import math

import jax
import jax.numpy as jnp
from jax import lax
from jax.experimental import pallas as pl
from jax.experimental.pallas import tpu as pltpu
from jax.experimental.pallas import tpu_sc as plsc

_HIGH = lax.Precision.HIGHEST
_QENC = 1 << 20


def _proj_body(xt_ref, w_ref, b_ref, qf_ref, p_ref, qs_ref):
    s, bsz, d = xt_ref.shape
    x2 = xt_ref[...].reshape(s * bsz, d)
    qf = lax.dot_general(x2, w_ref[...], (((1,), (1,)), ((), ())),
                         precision=_HIGH) + b_ref[...]
    qf_ref[...] = qf
    p = jnp.mean(qf.reshape(s, bsz, d), axis=1)
    p_ref[...] = p
    qs_ref[...] = lax.dot_general(qf, p, (((1,), (1,)), ((), ())),
                                  precision=_HIGH)


def _sims_body(m_slots, rb, mem_ref, p_ref, o_ref):
    i = pl.program_id(0)
    sims = lax.dot_general(mem_ref[...], p_ref[...],
                           (((1,), (1,)), ((), ())), precision=_HIGH)
    row = i * rb + lax.broadcasted_iota(jnp.int32, sims.shape, 0)
    o_ref[...] = jnp.where(row < m_slots, sims, -jnp.inf)


def _insert(state, v, iv):
    m0, m1, m2, m3, i0, i1, i2, i3 = state
    g0 = v > m0
    g1 = v > m1
    g2 = v > m2
    g3 = v > m3
    n0 = jnp.where(g0, v, m0)
    n1 = jnp.where(g0, m0, jnp.where(g1, v, m1))
    n2 = jnp.where(g1, m1, jnp.where(g2, v, m2))
    n3 = jnp.where(g2, m2, jnp.where(g3, v, m3))
    j0 = jnp.where(g0, iv, i0)
    j1 = jnp.where(g0, i0, jnp.where(g1, iv, i1))
    j2 = jnp.where(g1, i1, jnp.where(g2, iv, i2))
    j3 = jnp.where(g2, i2, jnp.where(g3, iv, i3))
    return (n0, n1, n2, n3, j0, j1, j2, j3)


def _top4_init():
    neg = jnp.full((16,), -jnp.inf, jnp.float32)
    nil = jnp.full((16,), -1, jnp.int32)
    return (neg, neg, neg, neg, nil, nil, nil, nil)


def _c1_body(rows_per_w, nw, sims_hbm, pv_hbm, pi_hbm, buf, cv, ci):
    wid = lax.axis_index("s") * 2 + lax.axis_index("c")
    prows = rows_per_w // 8
    base = wid * rows_per_w
    pltpu.sync_copy(sims_hbm.at[pl.ds(wid * prows, prows)], buf)
    t32 = lax.iota(jnp.int32, 16) * 32
    neg = jnp.full((16,), -jnp.inf, jnp.float32)

    def step(pr, state):
        for j in range(8):
            v = buf[pr, pl.ds(j * 16, 16)]
            gv = jnp.broadcast_to(base + pr * 8 + j, (16,))
            velig = jnp.where(gv >= t32, v, neg)
            state = _insert(state, velig, gv)
        return state

    m0, m1, m2, m3, i0, i1, i2, i3 = lax.fori_loop(
        0, prows, step, _top4_init())
    cv[0] = m0
    cv[1] = m1
    cv[2] = m2
    cv[3] = m3
    ci[0] = i0
    ci[1] = i1
    ci[2] = i2
    ci[3] = i3
    pltpu.sync_copy(cv, pv_hbm.at[wid])
    pltpu.sync_copy(ci, pi_hbm.at[wid])


def _c2_body(nw, qrows, pv_hbm, pi_hbm, qs_hbm, mem_hbm, qf_hbm,
             km_hbm, kq_hbm, isq_hbm, pvb, pib, qsb, gbuf, sbuf, sem):
    wid = lax.axis_index("s") * 2 + lax.axis_index("c")

    @pl.when(wid == 0)
    def _():
        pltpu.sync_copy(pv_hbm, pvb)
        pltpu.sync_copy(pi_hbm, pib)
        pltpu.sync_copy(qs_hbm, qsb)
        t32 = lax.iota(jnp.int32, 16) * 32
        neg = jnp.full((16,), -jnp.inf, jnp.float32)

        state = _top4_init()
        for pk in range((nw * 4) // 8):
            for j8 in range(8):
                v = pvb[pk, pl.ds(j8 * 16, 16)]
                iv = pib[pk, pl.ds(j8 * 16, 16)]
                state = _insert(state, v, iv)

        def merge_q(r8, state):
            for j8 in range(8):
                v = qsb[r8, pl.ds(j8 * 16, 16)]
                rv = jnp.broadcast_to(r8 * 8 + j8, (16,))
                velig = jnp.where(rv < t32, v, neg)
                state = _insert(state, velig, rv + _QENC)
            return state

        state = lax.fori_loop(0, qrows // 8, merge_q, state)
        idxs = state[4:]
        copies = []
        for j in range(4):
            ij = idxs[j]
            isq = ij >= _QENC
            im = jnp.where(isq, 0, ij)
            iq = jnp.where(isq, ij - _QENC, 0)
            copies.append(pltpu.async_copy(mem_hbm.at[im], gbuf.at[j], sem))
            copies.append(pltpu.async_copy(qf_hbm.at[iq], gbuf.at[4 + j], sem))
            sbuf[j] = jnp.where(isq, 1.0, 0.0).astype(jnp.float32)
        for c in copies:
            c.wait()
        pltpu.sync_copy(gbuf.at[pl.ds(0, 4)], km_hbm)
        pltpu.sync_copy(gbuf.at[pl.ds(4, 4)], kq_hbm)
        pltpu.sync_copy(sbuf, isq_hbm)


def _attn_body(q_ref, km_ref, kq_ref, isq_ref, o_ref):
    d = q_ref.shape[-1]
    kmv = km_ref[...]
    kqv = kq_ref[...]
    w = isq_ref[...][:, :, None]
    kv = kmv + (kqv - kmv) * w
    q = q_ref[...]
    scores = lax.dot_general(q, kv, (((2,), (2,)), ((0,), (1,))),
                             precision=_HIGH) / jnp.sqrt(jnp.float32(d))
    m = jnp.max(scores, axis=-1, keepdims=True)
    e = jnp.exp(scores - m)
    sm = e / jnp.sum(e, axis=-1, keepdims=True)
    o_ref[...] = lax.dot_general(sm, kv, (((2,), (0,)), ((0,), (1,))),
                                 precision=_HIGH)


def kernel(x, W, b, mem):
    bsz, s, d = x.shape
    m_slots = mem.shape[0]
    qrows = bsz * s
    f32 = jnp.float32

    xt = jnp.swapaxes(x, 0, 1)

    qf, p, qs = pl.pallas_call(
        _proj_body,
        out_shape=(
            jax.ShapeDtypeStruct((qrows, d), f32),
            jax.ShapeDtypeStruct((s, d), f32),
            jax.ShapeDtypeStruct((qrows, s), f32),
        ),
    )(xt, W, b.reshape(1, d))

    nw = 32
    rows_per_w = -(-m_slots // nw)
    rows_per_w = -(-rows_per_w // 64) * 64
    m_pad = nw * rows_per_w
    rb = rows_per_w
    sims = pl.pallas_call(
        functools.partial(_sims_body, m_slots, rb),
        grid=(m_pad // rb,),
        in_specs=[
            pl.BlockSpec((rb, d), lambda i: (i, 0)),
            pl.BlockSpec((s, d), lambda i: (0, 0)),
        ],
        out_specs=pl.BlockSpec((rb, s), lambda i: (i, 0)),
        out_shape=jax.ShapeDtypeStruct((m_pad, s), f32),
    )(mem, p)

    mesh = plsc.VectorSubcoreMesh(core_axis_name="c", subcore_axis_name="s")

    sims_p = sims.reshape(m_pad * s // 128, 128)
    pv, pi = pl.kernel(
        functools.partial(_c1_body, rows_per_w, nw),
        out_type=(
            jax.ShapeDtypeStruct((nw, 4, s), f32),
            jax.ShapeDtypeStruct((nw, 4, s), jnp.int32),
        ),
        mesh=mesh,
        scratch_types=[
            pltpu.VMEM((rows_per_w * s // 128, 128), f32),
            pltpu.VMEM((4, s), f32),
            pltpu.VMEM((4, s), jnp.int32),
        ],
    )(sims_p)

    km, kq, isq = pl.kernel(
        functools.partial(_c2_body, nw, qrows),
        out_type=(
            jax.ShapeDtypeStruct((4, s, d), f32),
            jax.ShapeDtypeStruct((4, s, d), f32),
            jax.ShapeDtypeStruct((4, s), f32),
        ),
        mesh=mesh,
        scratch_types=[
            pltpu.VMEM((nw * 4 * s // 128, 128), f32),
            pltpu.VMEM((nw * 4 * s // 128, 128), jnp.int32),
            pltpu.VMEM((qrows * s // 128, 128), f32),
            pltpu.VMEM((8, s, d), f32),
            pltpu.VMEM((4, s), f32),
            pltpu.SemaphoreType.DMA,
        ],
    )(pv.reshape(nw * 4 * s // 128, 128), pi.reshape(nw * 4 * s // 128, 128),
      qs.reshape(qrows * s // 128, 128), mem, qf)

    out_sm = pl.pallas_call(
        _attn_body,
        out_shape=jax.ShapeDtypeStruct((s, bsz, d), f32),
    )(qf.reshape(s, bsz, d), km, kq, isq)

    return jnp.swapaxes(out_sm, 0, 1)


# speedup vs baseline: 24.0184x; 1.0104x over previous
import functools
import math

import jax
import jax.numpy as jnp
from jax import lax
from jax.experimental import pallas as pl
from jax.experimental.pallas import tpu as pltpu
from jax.experimental.pallas import tpu_sc as plsc

_HIGH = lax.Precision.HIGHEST
_QENC = 1 << 20


def _ab_body(m_slots, rb, xt_ref, w_ref, b_ref, mem_ref,
             qf_ref, qs_ref, sims_ref, p_scr):
    i = pl.program_id(0)

    @pl.when(i == 0)
    def _():
        s, bsz, d = xt_ref.shape
        x2 = xt_ref[...].reshape(s * bsz, d)
        qf = lax.dot_general(x2, w_ref[...], (((1,), (1,)), ((), ())),
                             precision=_HIGH) + b_ref[...]
        qf_ref[...] = qf
        p = jnp.mean(qf.reshape(s, bsz, d), axis=1)
        p_scr[...] = p
        qs_ref[...] = lax.dot_general(qf, p, (((1,), (1,)), ((), ())),
                                      precision=_HIGH)

    sims = lax.dot_general(mem_ref[...], p_scr[...],
                           (((1,), (1,)), ((), ())), precision=_HIGH)
    row = i * rb + lax.broadcasted_iota(jnp.int32, sims.shape, 0)
    sims_ref[...] = jnp.where(row < m_slots, sims, -jnp.inf)


def _insert(state, v, iv):
    m0, m1, m2, m3, i0, i1, i2, i3 = state
    g0 = v > m0
    g1 = v > m1
    g2 = v > m2
    g3 = v > m3
    n0 = jnp.where(g0, v, m0)
    n1 = jnp.where(g0, m0, jnp.where(g1, v, m1))
    n2 = jnp.where(g1, m1, jnp.where(g2, v, m2))
    n3 = jnp.where(g2, m2, jnp.where(g3, v, m3))
    j0 = jnp.where(g0, iv, i0)
    j1 = jnp.where(g0, i0, jnp.where(g1, iv, i1))
    j2 = jnp.where(g1, i1, jnp.where(g2, iv, i2))
    j3 = jnp.where(g2, i2, jnp.where(g3, iv, i3))
    return (n0, n1, n2, n3, j0, j1, j2, j3)


def _top4_init():
    neg = jnp.full((16,), -jnp.inf, jnp.float32)
    nil = jnp.full((16,), -1, jnp.int32)
    return (neg, neg, neg, neg, nil, nil, nil, nil)


def _c1_body(rows_per_w, nw, qrows, sims_hbm, qs_hbm, pv_hbm, pi_hbm,
             buf, qbuf, cv, ci):
    wid = lax.axis_index("s") * 2 + lax.axis_index("c")
    prows = rows_per_w // 8
    base = wid * rows_per_w
    pltpu.sync_copy(sims_hbm.at[pl.ds(wid * prows, prows)], buf)
    pltpu.sync_copy(qs_hbm, qbuf)
    t32 = lax.iota(jnp.int32, 16) * 32
    neg = jnp.full((16,), -jnp.inf, jnp.float32)
    mrows = min(prows, (16 * 32) // 8)

    def step_masked(pr, state):
        for j in range(8):
            v = buf[pr, pl.ds(j * 16, 16)]
            gv = jnp.broadcast_to(base + pr * 8 + j, (16,))
            velig = jnp.where(gv >= t32, v, neg)
            state = _insert(state, velig, gv)
        return state

    def step_plain(pr, state):
        for j in range(8):
            v = buf[pr, pl.ds(j * 16, 16)]
            gv = jnp.broadcast_to(base + pr * 8 + j, (16,))
            state = _insert(state, v, gv)
        return state

    state = lax.fori_loop(0, mrows, step_masked, _top4_init())
    state = lax.fori_loop(mrows, prows, step_plain, state)

    qpp = (qrows // nw) // 8
    for pq in range(qpp):
        for j in range(8):
            r = wid * (qrows // nw) + pq * 8 + j
            v = qbuf[wid * qpp + pq, pl.ds(j * 16, 16)]
            rv = jnp.broadcast_to(r, (16,))
            velig = jnp.where(rv < t32, v, neg)
            state = _insert(state, velig, rv + _QENC)

    m0, m1, m2, m3, i0, i1, i2, i3 = state
    cv[0] = m0
    cv[1] = m1
    cv[2] = m2
    cv[3] = m3
    ci[0] = i0
    ci[1] = i1
    ci[2] = i2
    ci[3] = i3
    pltpu.sync_copy(cv, pv_hbm.at[wid])
    pltpu.sync_copy(ci, pi_hbm.at[wid])


def _c2_body(nw, pv_hbm, pi_hbm, mem_hbm, qf_hbm,
             km_hbm, kq_hbm, isq_hbm, pvb, pib, gbuf, sbuf, sem):
    wid = lax.axis_index("s") * 2 + lax.axis_index("c")

    @pl.when(wid == 0)
    def _():
        pltpu.sync_copy(pv_hbm, pvb)
        pltpu.sync_copy(pi_hbm, pib)

        state = _top4_init()
        for pk in range((nw * 4) // 8):
            for j8 in range(8):
                v = pvb[pk, pl.ds(j8 * 16, 16)]
                iv = pib[pk, pl.ds(j8 * 16, 16)]
                state = _insert(state, v, iv)
        idxs = state[4:]
        copies = []
        for j in range(4):
            ij = idxs[j]
            isq = ij >= _QENC
            im = jnp.where(isq, 0, ij)
            iq = jnp.where(isq, ij - _QENC, 0)
            copies.append(pltpu.async_copy(mem_hbm.at[im], gbuf.at[j], sem))
            copies.append(pltpu.async_copy(qf_hbm.at[iq], gbuf.at[4 + j], sem))
            sbuf[j] = jnp.where(isq, 1.0, 0.0).astype(jnp.float32)
        for c in copies:
            c.wait()
        pltpu.sync_copy(gbuf.at[pl.ds(0, 4)], km_hbm)
        pltpu.sync_copy(gbuf.at[pl.ds(4, 4)], kq_hbm)
        pltpu.sync_copy(sbuf, isq_hbm)


def _attn_body(q_ref, km_ref, kq_ref, isq_ref, o_ref):
    d = q_ref.shape[-1]
    kmv = km_ref[...]
    kqv = kq_ref[...]
    w = isq_ref[...][:, :, None]
    kv = kmv + (kqv - kmv) * w
    q = q_ref[...]
    scores = lax.dot_general(q, kv, (((2,), (2,)), ((0,), (1,))),
                             precision=_HIGH) / jnp.sqrt(jnp.float32(d))
    m = jnp.max(scores, axis=-1, keepdims=True)
    e = jnp.exp(scores - m)
    sm = e / jnp.sum(e, axis=-1, keepdims=True)
    o_ref[...] = lax.dot_general(sm, kv, (((2,), (0,)), ((0,), (1,))),
                                 precision=_HIGH)


def kernel(x, W, b, mem):
    bsz, s, d = x.shape
    m_slots = mem.shape[0]
    qrows = bsz * s
    f32 = jnp.float32

    xt = jnp.swapaxes(x, 0, 1)

    nw = 32
    rows_per_w = -(-m_slots // nw)
    rows_per_w = -(-rows_per_w // 64) * 64
    m_pad = nw * rows_per_w
    rb = rows_per_w
    qf, qs, sims = pl.pallas_call(
        functools.partial(_ab_body, m_slots, rb),
        grid=(m_pad // rb,),
        in_specs=[
            pl.BlockSpec((s, bsz, d), lambda i: (0, 0, 0)),
            pl.BlockSpec((d, d), lambda i: (0, 0)),
            pl.BlockSpec((1, d), lambda i: (0, 0)),
            pl.BlockSpec((rb, d), lambda i: (i, 0)),
        ],
        out_specs=(
            pl.BlockSpec((qrows, d), lambda i: (0, 0)),
            pl.BlockSpec((qrows, s), lambda i: (0, 0)),
            pl.BlockSpec((rb, s), lambda i: (i, 0)),
        ),
        out_shape=(
            jax.ShapeDtypeStruct((qrows, d), f32),
            jax.ShapeDtypeStruct((qrows, s), f32),
            jax.ShapeDtypeStruct((m_pad, s), f32),
        ),
        scratch_shapes=[pltpu.VMEM((s, d), f32)],
    )(xt, W, b.reshape(1, d), mem)

    mesh = plsc.VectorSubcoreMesh(core_axis_name="c", subcore_axis_name="s")

    sims_p = sims.reshape(m_pad * s // 128, 128)
    pv, pi = pl.kernel(
        functools.partial(_c1_body, rows_per_w, nw, qrows),
        out_type=(
            jax.ShapeDtypeStruct((nw, 4, s), f32),
            jax.ShapeDtypeStruct((nw, 4, s), jnp.int32),
        ),
        mesh=mesh,
        scratch_types=[
            pltpu.VMEM((rows_per_w * s // 128, 128), f32),
            pltpu.VMEM((qrows * s // 128, 128), f32),
            pltpu.VMEM((4, s), f32),
            pltpu.VMEM((4, s), jnp.int32),
        ],
    )(sims_p, qs.reshape(qrows * s // 128, 128))

    km, kq, isq = pl.kernel(
        functools.partial(_c2_body, nw),
        out_type=(
            jax.ShapeDtypeStruct((4, s, d), f32),
            jax.ShapeDtypeStruct((4, s, d), f32),
            jax.ShapeDtypeStruct((4, s), f32),
        ),
        mesh=mesh,
        scratch_types=[
            pltpu.VMEM((nw * 4 * s // 128, 128), f32),
            pltpu.VMEM((nw * 4 * s // 128, 128), jnp.int32),
            pltpu.VMEM((8, s, d), f32),
            pltpu.VMEM((4, s), f32),
            pltpu.SemaphoreType.DMA,
        ],
    )(pv.reshape(nw * 4 * s // 128, 128), pi.reshape(nw * 4 * s // 128, 128),
      mem, qf)

    out_sm = pl.pallas_call(
        _attn_body,
        out_shape=jax.ShapeDtypeStruct((s, bsz, d), f32),
    )(qf.reshape(s, bsz, d), km, kq, isq)

    return jnp.swapaxes(out_sm, 0, 1)


# speedup vs baseline: 32.4971x; 1.3530x over previous
import functools

import numpy as np

import jax
import jax.numpy as jnp
from jax import lax
from jax.experimental import pallas as pl
from jax.experimental.pallas import tpu as pltpu
from jax.experimental.pallas import tpu_sc as plsc

_HIGH = lax.Precision.HIGHEST
_QENC = 1 << 20


def _ab_body(m_slots, rb, xt_ref, w_ref, b_ref, mem_ref,
             qf_ref, qs_ref, sims_ref, p_scr):
    i = pl.program_id(0)

    @pl.when(i == 0)
    def _():
        s, bsz, d = xt_ref.shape
        x2 = xt_ref[...].reshape(s * bsz, d)
        qf = lax.dot_general(x2, w_ref[...], (((1,), (1,)), ((), ())),
                             precision=_HIGH) + b_ref[...]
        qf_ref[...] = qf
        p = jnp.mean(qf.reshape(s, bsz, d), axis=1)
        p_scr[...] = p
        qs_ref[...] = lax.dot_general(p, qf, (((1,), (1,)), ((), ())),
                                      precision=_HIGH)

    sims = lax.dot_general(p_scr[...], mem_ref[...],
                           (((1,), (1,)), ((), ())), precision=_HIGH)
    row = i * rb + lax.broadcasted_iota(jnp.int32, sims.shape, 1)
    sims_ref[...] = jnp.where(row < m_slots, sims, -jnp.inf)


def _insert(state, v, iv):
    m0, m1, m2, m3, i0, i1, i2, i3 = state
    g0 = v > m0
    g1 = v > m1
    g2 = v > m2
    g3 = v > m3
    n0 = jnp.where(g0, v, m0)
    n1 = jnp.where(g0, m0, jnp.where(g1, v, m1))
    n2 = jnp.where(g1, m1, jnp.where(g2, v, m2))
    n3 = jnp.where(g2, m2, jnp.where(g3, v, m3))
    j0 = jnp.where(g0, iv, i0)
    j1 = jnp.where(g0, i0, jnp.where(g1, iv, i1))
    j2 = jnp.where(g1, i1, jnp.where(g2, iv, i2))
    j3 = jnp.where(g2, i2, jnp.where(g3, iv, i3))
    return (n0, n1, n2, n3, j0, j1, j2, j3)


def _take16(x, idx):
    dnums = lax.GatherDimensionNumbers(
        offset_dims=(), collapsed_slice_dims=(0,), start_index_map=(0,))
    return lax.gather(x, idx[:, None], dnums, (1,),
                      mode=lax.GatherScatterMode.PROMISE_IN_BOUNDS)


def _allmax(v, i, lane):
    l = lane
    for d in (8, 4, 2, 1):
        pv = _take16(v, lane ^ d)
        pi = _take16(i, lane ^ d)
        pn = _take16(l, lane ^ d)
        m = pv > v
        v = jnp.where(m, pv, v)
        i = jnp.where(m, pi, i)
        l = jnp.where(m, pn, l)
    return v, i, l


def _top4_init():
    neg = jnp.full((16,), -jnp.inf, jnp.float32)
    nil = jnp.full((16,), -1, jnp.int32)
    return (neg, neg, neg, neg, nil, nil, nil, nil)


def _c1_body(rows_per_w, nw, qrows, sims_hbm, qs_hbm, pv_hbm, pi_hbm,
             buf, qbuf, cv, ci):
    wid = lax.axis_index("s") * 2 + lax.axis_index("c")
    base = wid * rows_per_w
    pltpu.sync_copy(sims_hbm.at[:, pl.ds(base, rows_per_w)], buf)
    pltpu.sync_copy(qs_hbm, qbuf)
    lane = lax.iota(jnp.int32, 16)
    neg = jnp.full((16,), -jnp.inf, jnp.float32)
    nvr = rows_per_w // 16
    nmask = min(nvr, 512 // 16)
    out_v = [neg] * 4
    out_i = [jnp.full((16,), -1, jnp.int32)] * 4

    for t in range(16):
        thr = jnp.full((16,), 32 * t, jnp.int32)

        def ins_masked(vi, st):
            v = buf[t, pl.ds(vi * 16, 16)]
            iv = lane + (base + vi * 16)
            velig = jnp.where(iv >= thr, v, neg)
            return _insert(st, velig, iv)

        def ins_plain(vi, st):
            v = buf[t, pl.ds(vi * 16, 16)]
            iv = lane + (base + vi * 16)
            return _insert(st, v, iv)

        st = lax.fori_loop(0, nmask, ins_masked, _top4_init())
        st = lax.fori_loop(nmask, nvr, ins_plain, st)

        vq = qbuf[t, pl.ds(wid * 16, 16)]
        ivq = lane + wid * 16
        veq = jnp.where(ivq < thr, vq, neg)
        st = _insert(st, veq, ivq + _QENC)

        vl = [st[0], st[1], st[2], st[3]]
        il = [st[4], st[5], st[6], st[7]]
        stepmask = lane == t
        for k in range(4):
            gv, gi, gl = _allmax(vl[0], il[0], lane)
            out_v[k] = jnp.where(stepmask, gv, out_v[k])
            out_i[k] = jnp.where(stepmask, gi, out_i[k])
            if k < 3:
                sel = lane == gl
                for r in range(3):
                    vl[r] = jnp.where(sel, vl[r + 1], vl[r])
                    il[r] = jnp.where(sel, il[r + 1], il[r])
                vl[3] = jnp.where(sel, neg, vl[3])

    m0, m1, m2, m3 = out_v
    i0, i1, i2, i3 = out_i
    cv[0] = m0
    cv[1] = m1
    cv[2] = m2
    cv[3] = m3
    ci[0] = i0
    ci[1] = i1
    ci[2] = i2
    ci[3] = i3
    pltpu.sync_copy(cv, pv_hbm.at[wid])
    pltpu.sync_copy(ci, pi_hbm.at[wid])


def _c2_body(nw, pv_hbm, pi_hbm, mem_hbm, qf_hbm,
             km_hbm, kq_hbm, isq_hbm, pvb, pib, gbuf, sbuf, sem):
    wid = lax.axis_index("s") * 2 + lax.axis_index("c")

    @pl.when(wid == 0)
    def _():
        pltpu.sync_copy(pv_hbm, pvb)
        pltpu.sync_copy(pi_hbm, pib)

        state = _top4_init()
        for pk in range((nw * 4) // 8):
            for j8 in range(8):
                v = pvb[pk, pl.ds(j8 * 16, 16)]
                iv = pib[pk, pl.ds(j8 * 16, 16)]
                state = _insert(state, v, iv)
        idxs = state[4:]
        copies = []
        for j in range(4):
            ij = idxs[j]
            isq = ij >= _QENC
            im = jnp.where(isq, 0, ij)
            iq = jnp.where(isq, ij - _QENC, 0)
            copies.append(pltpu.async_copy(mem_hbm.at[im], gbuf.at[j], sem))
            copies.append(pltpu.async_copy(qf_hbm.at[iq], gbuf.at[4 + j], sem))
            sbuf[j] = jnp.where(isq, 1.0, 0.0).astype(jnp.float32)
        for c in copies:
            c.wait()
        pltpu.sync_copy(gbuf.at[pl.ds(0, 4)], km_hbm)
        pltpu.sync_copy(gbuf.at[pl.ds(4, 4)], kq_hbm)
        pltpu.sync_copy(sbuf, isq_hbm)


def _attn_body(q_ref, km_ref, kq_ref, isq_ref, o_ref):
    d = q_ref.shape[-1]
    kmv = km_ref[...]
    kqv = kq_ref[...]
    w = isq_ref[...][:, :, None]
    kv = kmv + (kqv - kmv) * w
    q = q_ref[...]
    scores = lax.dot_general(q, kv, (((2,), (2,)), ((0,), (1,))),
                             precision=_HIGH) / jnp.sqrt(jnp.float32(d))
    m = jnp.max(scores, axis=-1, keepdims=True)
    e = jnp.exp(scores - m)
    sm = e / jnp.sum(e, axis=-1, keepdims=True)
    o_ref[...] = lax.dot_general(sm, kv, (((2,), (0,)), ((0,), (1,))),
                                 precision=_HIGH)


def kernel(x, W, b, mem):
    bsz, s, d = x.shape
    m_slots = mem.shape[0]
    qrows = bsz * s
    f32 = jnp.float32

    xt = jnp.swapaxes(x, 0, 1)

    nw = 32
    rows_per_w = -(-m_slots // nw)
    rows_per_w = -(-rows_per_w // 128) * 128
    m_pad = nw * rows_per_w
    rb = rows_per_w
    qf, qs, sims = pl.pallas_call(
        functools.partial(_ab_body, m_slots, rb),
        grid=(m_pad // rb,),
        in_specs=[
            pl.BlockSpec((s, bsz, d), lambda i: (0, 0, 0)),
            pl.BlockSpec((d, d), lambda i: (0, 0)),
            pl.BlockSpec((1, d), lambda i: (0, 0)),
            pl.BlockSpec((rb, d), lambda i: (i, 0)),
        ],
        out_specs=(
            pl.BlockSpec((qrows, d), lambda i: (0, 0)),
            pl.BlockSpec((s, qrows), lambda i: (0, 0)),
            pl.BlockSpec((s, rb), lambda i: (0, i)),
        ),
        out_shape=(
            jax.ShapeDtypeStruct((qrows, d), f32),
            jax.ShapeDtypeStruct((s, qrows), f32),
            jax.ShapeDtypeStruct((s, m_pad), f32),
        ),
        scratch_shapes=[pltpu.VMEM((s, d), f32)],
    )(xt, W, b.reshape(1, d), mem)

    mesh = plsc.VectorSubcoreMesh(core_axis_name="c", subcore_axis_name="s")

    pv, pi = pl.kernel(
        functools.partial(_c1_body, rows_per_w, nw, qrows),
        out_type=(
            jax.ShapeDtypeStruct((nw, 4, s), f32),
            jax.ShapeDtypeStruct((nw, 4, s), jnp.int32),
        ),
        mesh=mesh,
        scratch_types=[
            pltpu.VMEM((s, rows_per_w), f32),
            pltpu.VMEM((s, qrows), f32),
            pltpu.VMEM((4, s), f32),
            pltpu.VMEM((4, s), jnp.int32),
        ],
    )(sims, qs)

    km, kq, isq = pl.kernel(
        functools.partial(_c2_body, nw),
        out_type=(
            jax.ShapeDtypeStruct((4, s, d), f32),
            jax.ShapeDtypeStruct((4, s, d), f32),
            jax.ShapeDtypeStruct((4, s), f32),
        ),
        mesh=mesh,
        scratch_types=[
            pltpu.VMEM((nw * 4 * s // 128, 128), f32),
            pltpu.VMEM((nw * 4 * s // 128, 128), jnp.int32),
            pltpu.VMEM((8, s, d), f32),
            pltpu.VMEM((4, s), f32),
            pltpu.SemaphoreType.DMA,
        ],
    )(pv.reshape(nw * 4 * s // 128, 128), pi.reshape(nw * 4 * s // 128, 128),
      mem, qf)

    out_sm = pl.pallas_call(
        _attn_body,
        out_shape=jax.ShapeDtypeStruct((s, bsz, d), f32),
    )(qf.reshape(s, bsz, d), km, kq, isq)

    return jnp.swapaxes(out_sm, 0, 1)


# speedup vs baseline: 38.6691x; 1.1899x over previous
import functools

import numpy as np

import jax
import jax.numpy as jnp
from jax import lax
from jax.experimental import pallas as pl
from jax.experimental.pallas import tpu as pltpu
from jax.experimental.pallas import tpu_sc as plsc

_HIGH = lax.Precision.HIGHEST
_QENC = 1 << 20


def _ab_body(m_slots, rb, xt_ref, w_ref, b_ref, mem_ref,
             qf_ref, qs_ref, sims_ref, p_scr):
    i = pl.program_id(0)

    @pl.when(i == 0)
    def _():
        s, bsz, d = xt_ref.shape
        x2 = xt_ref[...].reshape(s * bsz, d)
        qf = lax.dot_general(x2, w_ref[...], (((1,), (1,)), ((), ())),
                             precision=_HIGH) + b_ref[...]
        qf_ref[...] = qf
        p = jnp.mean(qf.reshape(s, bsz, d), axis=1)
        p_scr[...] = p
        qs_ref[...] = lax.dot_general(p, qf, (((1,), (1,)), ((), ())),
                                      precision=_HIGH)

    sims = lax.dot_general(p_scr[...], mem_ref[...],
                           (((1,), (1,)), ((), ())))
    row = i * rb + lax.broadcasted_iota(jnp.int32, sims.shape, 1)
    sims_ref[...] = jnp.where(row < m_slots, sims, -jnp.inf)


def _insert(state, v, iv):
    m0, m1, m2, m3, i0, i1, i2, i3 = state
    g0 = v > m0
    g1 = v > m1
    g2 = v > m2
    g3 = v > m3
    n0 = jnp.where(g0, v, m0)
    n1 = jnp.where(g0, m0, jnp.where(g1, v, m1))
    n2 = jnp.where(g1, m1, jnp.where(g2, v, m2))
    n3 = jnp.where(g2, m2, jnp.where(g3, v, m3))
    j0 = jnp.where(g0, iv, i0)
    j1 = jnp.where(g0, i0, jnp.where(g1, iv, i1))
    j2 = jnp.where(g1, i1, jnp.where(g2, iv, i2))
    j3 = jnp.where(g2, i2, jnp.where(g3, iv, i3))
    return (n0, n1, n2, n3, j0, j1, j2, j3)


def _take16(x, idx):
    dnums = lax.GatherDimensionNumbers(
        offset_dims=(), collapsed_slice_dims=(0,), start_index_map=(0,))
    return lax.gather(x, idx[:, None], dnums, (1,),
                      mode=lax.GatherScatterMode.PROMISE_IN_BOUNDS)


def _allmax(v, i, lane):
    l = lane
    for d in (8, 4, 2, 1):
        pv = _take16(v, lane ^ d)
        pi = _take16(i, lane ^ d)
        pn = _take16(l, lane ^ d)
        m = pv > v
        v = jnp.where(m, pv, v)
        i = jnp.where(m, pi, i)
        l = jnp.where(m, pn, l)
    return v, i, l


def _top4_init():
    neg = jnp.full((16,), -jnp.inf, jnp.float32)
    nil = jnp.full((16,), -1, jnp.int32)
    return (neg, neg, neg, neg, nil, nil, nil, nil)


def _c1_body(rows_per_w, nw, qrows, sims_hbm, qs_hbm, pv_hbm, pi_hbm,
             buf, qbuf, cv, ci):
    wid = lax.axis_index("s") * 2 + lax.axis_index("c")
    base = wid * rows_per_w
    pltpu.sync_copy(sims_hbm.at[:, pl.ds(base, rows_per_w)], buf)
    pltpu.sync_copy(qs_hbm, qbuf)
    lane = lax.iota(jnp.int32, 16)
    neg = jnp.full((16,), -jnp.inf, jnp.float32)
    nvr = rows_per_w // 16
    nmask = min(nvr, 512 // 16)
    out_v = [neg] * 4
    out_i = [jnp.full((16,), -1, jnp.int32)] * 4

    for t in range(16):
        thr = jnp.full((16,), 32 * t, jnp.int32)

        def ins_masked(vi, st):
            v = buf[t, pl.ds(vi * 16, 16)]
            iv = lane + (base + vi * 16)
            velig = jnp.where(iv >= thr, v, neg)
            return _insert(st, velig, iv)

        def ins_plain(vi, st):
            v = buf[t, pl.ds(vi * 16, 16)]
            iv = lane + (base + vi * 16)
            return _insert(st, v, iv)

        st = lax.fori_loop(0, nmask, ins_masked, _top4_init())
        st = lax.fori_loop(nmask, nvr, ins_plain, st)

        vq = qbuf[t, pl.ds(wid * 16, 16)]
        ivq = lane + wid * 16
        veq = jnp.where(ivq < thr, vq, neg)
        st = _insert(st, veq, ivq + _QENC)

        vl = [st[0], st[1], st[2], st[3]]
        il = [st[4], st[5], st[6], st[7]]
        stepmask = lane == t
        for k in range(4):
            gv, gi, gl = _allmax(vl[0], il[0], lane)
            out_v[k] = jnp.where(stepmask, gv, out_v[k])
            out_i[k] = jnp.where(stepmask, gi, out_i[k])
            if k < 3:
                sel = lane == gl
                for r in range(3):
                    vl[r] = jnp.where(sel, vl[r + 1], vl[r])
                    il[r] = jnp.where(sel, il[r + 1], il[r])
                vl[3] = jnp.where(sel, neg, vl[3])

    m0, m1, m2, m3 = out_v
    i0, i1, i2, i3 = out_i
    cv[0] = m0
    cv[1] = m1
    cv[2] = m2
    cv[3] = m3
    ci[0] = i0
    ci[1] = i1
    ci[2] = i2
    ci[3] = i3
    pltpu.sync_copy(cv, pv_hbm.at[wid])
    pltpu.sync_copy(ci, pi_hbm.at[wid])


def _c2_body(nw, pv_hbm, pi_hbm, mem_hbm, qf_hbm,
             km_hbm, kq_hbm, isq_hbm, pvb, pib, gbuf, sbuf, sem):
    wid = lax.axis_index("s") * 2 + lax.axis_index("c")

    @pl.when(wid == 0)
    def _():
        pltpu.sync_copy(pv_hbm, pvb)
        pltpu.sync_copy(pi_hbm, pib)

        state = _top4_init()
        for pk in range((nw * 4) // 8):
            for j8 in range(8):
                v = pvb[pk, pl.ds(j8 * 16, 16)]
                iv = pib[pk, pl.ds(j8 * 16, 16)]
                state = _insert(state, v, iv)
        idxs = state[4:]
        copies = []
        for j in range(4):
            ij = idxs[j]
            isq = ij >= _QENC
            im = jnp.where(isq, 0, ij)
            iq = jnp.where(isq, ij - _QENC, 0)
            copies.append(pltpu.async_copy(mem_hbm.at[im], gbuf.at[j], sem))
            copies.append(pltpu.async_copy(qf_hbm.at[iq], gbuf.at[4 + j], sem))
            sbuf[j] = jnp.where(isq, 1.0, 0.0).astype(jnp.float32)
        for c in copies:
            c.wait()
        pltpu.sync_copy(gbuf.at[pl.ds(0, 4)], km_hbm)
        pltpu.sync_copy(gbuf.at[pl.ds(4, 4)], kq_hbm)
        pltpu.sync_copy(sbuf, isq_hbm)


def _attn_body(q_ref, km_ref, kq_ref, isq_ref, o_ref):
    d = q_ref.shape[-1]
    kmv = km_ref[...]
    kqv = kq_ref[...]
    w = isq_ref[...][:, :, None]
    kv = kmv + (kqv - kmv) * w
    q = q_ref[...]
    scores = lax.dot_general(q, kv, (((2,), (2,)), ((0,), (1,))),
                             precision=_HIGH) / jnp.sqrt(jnp.float32(d))
    m = jnp.max(scores, axis=-1, keepdims=True)
    e = jnp.exp(scores - m)
    sm = e / jnp.sum(e, axis=-1, keepdims=True)
    o_ref[...] = lax.dot_general(sm, kv, (((2,), (0,)), ((0,), (1,))),
                                 precision=_HIGH)


def kernel(x, W, b, mem):
    bsz, s, d = x.shape
    m_slots = mem.shape[0]
    qrows = bsz * s
    f32 = jnp.float32

    xt = jnp.swapaxes(x, 0, 1)

    nw = 32
    rows_per_w = -(-m_slots // nw)
    rows_per_w = -(-rows_per_w // 128) * 128
    m_pad = nw * rows_per_w
    rb = rows_per_w
    qf, qs, sims = pl.pallas_call(
        functools.partial(_ab_body, m_slots, rb),
        grid=(m_pad // rb,),
        in_specs=[
            pl.BlockSpec((s, bsz, d), lambda i: (0, 0, 0)),
            pl.BlockSpec((d, d), lambda i: (0, 0)),
            pl.BlockSpec((1, d), lambda i: (0, 0)),
            pl.BlockSpec((rb, d), lambda i: (i, 0)),
        ],
        out_specs=(
            pl.BlockSpec((qrows, d), lambda i: (0, 0)),
            pl.BlockSpec((s, qrows), lambda i: (0, 0)),
            pl.BlockSpec((s, rb), lambda i: (0, i)),
        ),
        out_shape=(
            jax.ShapeDtypeStruct((qrows, d), f32),
            jax.ShapeDtypeStruct((s, qrows), f32),
            jax.ShapeDtypeStruct((s, m_pad), f32),
        ),
        scratch_shapes=[pltpu.VMEM((s, d), f32)],
    )(xt, W, b.reshape(1, d), mem)

    mesh = plsc.VectorSubcoreMesh(core_axis_name="c", subcore_axis_name="s")

    pv, pi = pl.kernel(
        functools.partial(_c1_body, rows_per_w, nw, qrows),
        out_type=(
            jax.ShapeDtypeStruct((nw, 4, s), f32),
            jax.ShapeDtypeStruct((nw, 4, s), jnp.int32),
        ),
        mesh=mesh,
        scratch_types=[
            pltpu.VMEM((s, rows_per_w), f32),
            pltpu.VMEM((s, qrows), f32),
            pltpu.VMEM((4, s), f32),
            pltpu.VMEM((4, s), jnp.int32),
        ],
    )(sims, qs)

    km, kq, isq = pl.kernel(
        functools.partial(_c2_body, nw),
        out_type=(
            jax.ShapeDtypeStruct((4, s, d), f32),
            jax.ShapeDtypeStruct((4, s, d), f32),
            jax.ShapeDtypeStruct((4, s), f32),
        ),
        mesh=mesh,
        scratch_types=[
            pltpu.VMEM((nw * 4 * s // 128, 128), f32),
            pltpu.VMEM((nw * 4 * s // 128, 128), jnp.int32),
            pltpu.VMEM((8, s, d), f32),
            pltpu.VMEM((4, s), f32),
            pltpu.SemaphoreType.DMA,
        ],
    )(pv.reshape(nw * 4 * s // 128, 128), pi.reshape(nw * 4 * s // 128, 128),
      mem, qf)

    out_sm = pl.pallas_call(
        _attn_body,
        out_shape=jax.ShapeDtypeStruct((s, bsz, d), f32),
    )(qf.reshape(s, bsz, d), km, kq, isq)

    return jnp.swapaxes(out_sm, 0, 1)
